# baseline (speedup 1.0000x reference)
import functools
import math

import jax
import jax.numpy as jnp
import numpy as np
from jax import lax
from jax.experimental import pallas as pl
from jax.experimental.pallas import tpu as pltpu
from jax.experimental.pallas import tpu_sc as plsc

VOCAB = 1000000
D = 64
MAXLEN = 2048
B = 4096
S = 200
EPS = 1e-12

NC = 2
NS = 16
NW = NC * NS
N = B * S
PER_W = N // NW
CHUNK = 128
N_CHUNKS = PER_W // CHUNK


def _pe_over_8():
    position = np.arange(0, S, dtype=np.float32)[:, None]
    div_term = np.exp(
        np.arange(0, D, 2, dtype=np.float32) * (-math.log(10000.0) / D))
    pe = np.zeros((S, D), dtype=np.float32)
    pe[:, 0::2] = np.sin(position * div_term)
    pe[:, 1::2] = np.cos(position * div_term)
    return pe / 8.0


_PE8 = _pe_over_8()


def _sc_gather(ids_3d, W):
    mesh = plsc.VectorSubcoreMesh(core_axis_name="c", subcore_axis_name="s")

    @functools.partial(
        pl.kernel,
        mesh=mesh,
        compiler_params=pltpu.CompilerParams(use_tc_tiling_on_sc=False),
        out_type=jax.ShapeDtypeStruct((N, D), jnp.float32),
        scratch_types=[
            pltpu.VMEM((N_CHUNKS, CHUNK), jnp.int32),
            pltpu.VMEM((CHUNK, D), jnp.float32),
            pltpu.VMEM((CHUNK, D), jnp.float32),
            pltpu.SemaphoreType.DMA,
            pltpu.SemaphoreType.DMA,
        ],
    )
    def k(ids_hbm, w_hbm, out_hbm, idx_v, rows0, rows1, sem0, sem1):
        wid = lax.axis_index("s") * NC + lax.axis_index("c")
        base = wid * PER_W
        pltpu.sync_copy(ids_hbm.at[wid], idx_v)

        @pl.loop(0, N_CHUNKS, step=2)
        def _(c):
            cpa = pltpu.async_copy(w_hbm.at[idx_v.at[c]], rows0, sem0)
            cpb = pltpu.async_copy(w_hbm.at[idx_v.at[c + 1]], rows1, sem1)
            cpa.wait()
            pltpu.sync_copy(rows0, out_hbm.at[pl.ds(base + c * CHUNK, CHUNK)])
            cpb.wait()
            pltpu.sync_copy(
                rows1, out_hbm.at[pl.ds(base + (c + 1) * CHUNK, CHUNK)])

    return k(ids_3d, W)


BB = 128


def _ln_body(g_ref, pe_ref, gam_ref, bet_ref, o_ref):
    x = g_ref[...] + pe_ref[...][None, :, :]
    mu = jnp.mean(x, axis=-1, keepdims=True)
    xc = x - mu
    var = jnp.mean(xc * xc, axis=-1, keepdims=True)
    y = xc * lax.rsqrt(var + EPS / 64.0)
    y = y * gam_ref[...][None, :, :] + bet_ref[...][None, :, :]
    o_ref[...] = jnp.transpose(y, (1, 2, 0))


SS = 40


def _tc_layernorm(g, pe8, gamma, beta):
    return pl.pallas_call(
        _ln_body,
        grid=(B // BB, S // SS),
        in_specs=[
            pl.BlockSpec((BB, SS, D), lambda i, j: (i, j, 0)),
            pl.BlockSpec((SS, D), lambda i, j: (j, 0)),
            pl.BlockSpec((1, D), lambda i, j: (0, 0)),
            pl.BlockSpec((1, D), lambda i, j: (0, 0)),
        ],
        out_specs=pl.BlockSpec((SS, D, BB), lambda i, j: (j, 0, i)),
        out_shape=jax.ShapeDtypeStruct((S, D, B), jnp.float32),
    )(g, pe8, gamma, beta)


def kernel(input_ids, W, gamma, beta):
    ids_3d = input_ids.reshape(NW, N_CHUNKS, CHUNK).astype(jnp.int32)
    g = _sc_gather(ids_3d, W)
    g = g.reshape(B, S, D)
    pe8 = jnp.asarray(_PE8)
    y = _tc_layernorm(g, pe8, gamma.reshape(1, D), beta.reshape(1, D))
    return jnp.transpose(y, (2, 0, 1))


# speedup vs baseline: 1.3111x; 1.3111x over previous
import functools
import math

import jax
import jax.numpy as jnp
import numpy as np
from jax import lax
from jax.experimental import pallas as pl
from jax.experimental.pallas import tpu as pltpu
from jax.experimental.pallas import tpu_sc as plsc

VOCAB = 1000000
D = 64
MAXLEN = 2048
B = 4096
S = 200
EPS = 1e-12

NC = 2
NS = 16
NW = NC * NS
N = B * S
HB = B // 2
PER_W = N // NW
CHUNK = 128
N_CHUNKS = PER_W // CHUNK


def _pe_over_8():
    position = np.arange(0, S, dtype=np.float32)[:, None]
    div_term = np.exp(
        np.arange(0, D, 2, dtype=np.float32) * (-math.log(10000.0) / D))
    pe = np.zeros((S, D), dtype=np.float32)
    pe[:, 0::2] = np.sin(position * div_term)
    pe[:, 1::2] = np.cos(position * div_term)
    return pe / 8.0


_PE8 = _pe_over_8()

VB = 2048


def _conv_body(wt_ref, o_ref):
    x = wt_ref[...]
    xt = x.T.reshape(VB // 2, 2, D)
    o_ref[:, :D] = xt[:, 0, :]
    o_ref[:, D:] = xt[:, 1, :]


def _tc_repack(W):
    wt = W.T
    return pl.pallas_call(
        _conv_body,
        grid=(pl.cdiv(VOCAB, VB),),
        in_specs=[pl.BlockSpec((D, VB), lambda i: (0, i))],
        out_specs=pl.BlockSpec((VB // 2, 2 * D), lambda i: (i, 0)),
        out_shape=jax.ShapeDtypeStruct((VOCAB // 2, 2 * D), jnp.float32),
    )(wt)


def _sc_gather(ids_3d, w_lin):
    mesh = plsc.VectorSubcoreMesh(core_axis_name="c", subcore_axis_name="s")

    @functools.partial(
        pl.kernel,
        mesh=mesh,
        compiler_params=pltpu.CompilerParams(use_tc_tiling_on_sc=False),
        out_type=jax.ShapeDtypeStruct((N, D), jnp.float32),
        scratch_types=[
            pltpu.VMEM((N_CHUNKS, CHUNK), jnp.int32),
            pltpu.VMEM((CHUNK, D), jnp.float32),
            pltpu.VMEM((CHUNK, D), jnp.float32),
            pltpu.SemaphoreType.DMA,
            pltpu.SemaphoreType.DMA,
        ],
    )
    def k(ids_hbm, w_hbm, out_hbm, idx_v, rows0, rows1, sem0, sem1):
        wid = lax.axis_index("s") * NC + lax.axis_index("c")
        base = wid * PER_W
        pltpu.sync_copy(ids_hbm.at[wid], idx_v)

        @pl.loop(0, N_CHUNKS, step=2)
        def _(c):
            cpa = pltpu.async_copy(w_hbm.at[idx_v.at[c]], rows0, sem0)
            cpb = pltpu.async_copy(w_hbm.at[idx_v.at[c + 1]], rows1, sem1)
            cpa.wait()
            pltpu.sync_copy(rows0, out_hbm.at[pl.ds(base + c * CHUNK, CHUNK)])
            cpb.wait()
            pltpu.sync_copy(
                rows1, out_hbm.at[pl.ds(base + (c + 1) * CHUNK, CHUNK)])

    return k(ids_3d, w_lin)


SC7 = 8


def _ln_body(g_ref, pe2_ref, gam2_ref, bet2_ref, o_ref):
    x = g_ref[...] + pe2_ref[...][None, :, :]
    lane = lax.broadcasted_iota(jnp.int32, (1, 1, 2 * D), 2)
    m_a = (lane < D).astype(jnp.float32)
    s_all = jnp.sum(x, axis=-1, keepdims=True)
    s_a = jnp.sum(x * m_a, axis=-1, keepdims=True)
    mu = jnp.where(lane < D, s_a / D, (s_all - s_a) / D)
    xc = x - mu
    q = xc * xc
    q_all = jnp.sum(q, axis=-1, keepdims=True)
    q_a = jnp.sum(q * m_a, axis=-1, keepdims=True)
    var = jnp.where(lane < D, q_a / D, (q_all - q_a) / D)
    y = xc * lax.rsqrt(var + EPS / 64.0)
    y = y * gam2_ref[...][None, :, :] + bet2_ref[...][None, :, :]
    for st in range(SC7):
        t = y[:, st, :].T
        o_ref[st, :, :HB] = t[:D]
        o_ref[st, :, HB:] = t[D:]


def _tc_layernorm(g, pe2, gamma2, beta2):
    return pl.pallas_call(
        _ln_body,
        grid=(S // SC7,),
        in_specs=[
            pl.BlockSpec((HB, SC7, 2 * D), lambda j: (0, j, 0)),
            pl.BlockSpec((SC7, 2 * D), lambda j: (j, 0)),
            pl.BlockSpec((1, 2 * D), lambda j: (0, 0)),
            pl.BlockSpec((1, 2 * D), lambda j: (0, 0)),
        ],
        out_specs=pl.BlockSpec((SC7, D, B), lambda j: (j, 0, 0)),
        out_shape=jax.ShapeDtypeStruct((S, D, B), jnp.float32),
    )(g, pe2, gamma2, beta2)


def kernel(input_ids, W, gamma, beta):
    w2 = _tc_repack(W)
    w_lin = w2.reshape(-1).reshape(VOCAB, D)
    ids_pair = jnp.stack(
        [input_ids[:HB], input_ids[HB:]], axis=-1).astype(jnp.int32)
    ids_3d = ids_pair.reshape(NW, N_CHUNKS, CHUNK)
    g = _sc_gather(ids_3d, w_lin)
    g = g.reshape(-1).reshape(HB, S, 2 * D)
    pe2 = jnp.concatenate([_PE8, _PE8], axis=1)
    gam2 = jnp.tile(gamma.reshape(1, D), (1, 2))
    bet2 = jnp.tile(beta.reshape(1, D), (1, 2))
    y = _tc_layernorm(g, jnp.asarray(pe2), gam2, bet2)
    return jnp.transpose(y, (2, 0, 1))


# speedup vs baseline: 1.4893x; 1.1359x over previous
import functools
import math

import jax
import jax.numpy as jnp
import numpy as np
from jax import lax
from jax.experimental import pallas as pl
from jax.experimental.pallas import tpu as pltpu
from jax.experimental.pallas import tpu_sc as plsc

VOCAB = 1000000
D = 64
MAXLEN = 2048
B = 4096
S = 200
EPS = 1e-12

NC = 2
NS = 16
NW = NC * NS
N = B * S
HB = B // 2
PER_W = N // NW
CHUNK = 128
N_CHUNKS = PER_W // CHUNK


def _pe_over_8():
    position = np.arange(0, S, dtype=np.float32)[:, None]
    div_term = np.exp(
        np.arange(0, D, 2, dtype=np.float32) * (-math.log(10000.0) / D))
    pe = np.zeros((S, D), dtype=np.float32)
    pe[:, 0::2] = np.sin(position * div_term)
    pe[:, 1::2] = np.cos(position * div_term)
    return pe / 8.0


_PE8 = _pe_over_8()

VB = 2048


def _conv_body(wt_ref, o_ref):
    x = wt_ref[...]
    xt = x.T.reshape(VB // 2, 2, D)
    o_ref[:, :D] = xt[:, 0, :]
    o_ref[:, D:] = xt[:, 1, :]


def _tc_repack(W):
    wt = W.T
    return pl.pallas_call(
        _conv_body,
        grid=(pl.cdiv(VOCAB, VB),),
        in_specs=[pl.BlockSpec((D, VB), lambda i: (0, i))],
        out_specs=pl.BlockSpec((VB // 2, 2 * D), lambda i: (i, 0)),
        out_shape=jax.ShapeDtypeStruct((VOCAB // 2, 2 * D), jnp.float32),
    )(wt)


IB = HB // NW


def _sc_gather(input_ids, w_lin):
    mesh = plsc.VectorSubcoreMesh(core_axis_name="c", subcore_axis_name="s")

    @functools.partial(
        pl.kernel,
        mesh=mesh,
        compiler_params=pltpu.CompilerParams(
            use_tc_tiling_on_sc=False, needs_layout_passes=False),
        out_type=jax.ShapeDtypeStruct((N, D), jnp.float32),
        scratch_types=[
            pltpu.VMEM((IB * S,), jnp.int32),
            pltpu.VMEM((IB * S,), jnp.int32),
            pltpu.VMEM((N_CHUNKS, CHUNK), jnp.int32),
            pltpu.VMEM((CHUNK, D), jnp.float32),
            pltpu.VMEM((CHUNK, D), jnp.float32),
            pltpu.SemaphoreType.DMA,
            pltpu.SemaphoreType.DMA,
        ],
    )
    def k(ids_hbm, w_hbm, out_hbm, ids_a, ids_b, idx_v, rows0, rows1,
          sem0, sem1):
        wid = lax.axis_index("s") * NC + lax.axis_index("c")
        base = wid * PER_W

        @pl.loop(0, IB)
        def _(j):
            pltpu.sync_copy(ids_hbm.at[wid * IB + j],
                            ids_a.at[pl.ds(j * S, S)])
            pltpu.sync_copy(ids_hbm.at[HB + wid * IB + j],
                            ids_b.at[pl.ds(j * S, S)])

        iota2 = lax.iota(jnp.int32, 16) * 2

        @pl.loop(0, IB * S // 16)
        def _(v):
            q = v * 16
            pos = iota2 + (q * 2)
            row = lax.shift_right_logical(pos, 7)
            col = lax.bitwise_and(pos, 127)
            xa = ids_a[pl.ds(q, 16)]
            plsc.store_scatter(idx_v, [row, col], xa)
            pos1 = pos + 1
            row1 = lax.shift_right_logical(pos1, 7)
            col1 = lax.bitwise_and(pos1, 127)
            xb = ids_b[pl.ds(q, 16)]
            plsc.store_scatter(idx_v, [row1, col1], xb)

        @pl.loop(0, N_CHUNKS, step=2)
        def _(c):
            cpa = pltpu.async_copy(w_hbm.at[idx_v.at[c]], rows0, sem0)
            cpb = pltpu.async_copy(w_hbm.at[idx_v.at[c + 1]], rows1, sem1)
            cpa.wait()
            pltpu.sync_copy(rows0, out_hbm.at[pl.ds(base + c * CHUNK, CHUNK)])
            cpb.wait()
            pltpu.sync_copy(
                rows1, out_hbm.at[pl.ds(base + (c + 1) * CHUNK, CHUNK)])

    return k(input_ids, w_lin)


SC7 = 8


def _ln_body(g_ref, pe2_ref, gam2_ref, bet2_ref, o_ref):
    x = g_ref[...] + pe2_ref[...][None, :, :]
    lane = lax.broadcasted_iota(jnp.int32, (1, 1, 2 * D), 2)
    m_a = (lane < D).astype(jnp.float32)
    s_all = jnp.sum(x, axis=-1, keepdims=True)
    s_a = jnp.sum(x * m_a, axis=-1, keepdims=True)
    mu = jnp.where(lane < D, s_a / D, (s_all - s_a) / D)
    xc = x - mu
    q = xc * xc
    q_all = jnp.sum(q, axis=-1, keepdims=True)
    q_a = jnp.sum(q * m_a, axis=-1, keepdims=True)
    var = jnp.where(lane < D, q_a / D, (q_all - q_a) / D)
    y = xc * lax.rsqrt(var + EPS / 64.0)
    y = y * gam2_ref[...][None, :, :] + bet2_ref[...][None, :, :]
    for st in range(SC7):
        t = y[:, st, :].T
        o_ref[st, :, :HB] = t[:D]
        o_ref[st, :, HB:] = t[D:]


def _tc_layernorm(g, pe2, gamma2, beta2):
    return pl.pallas_call(
        _ln_body,
        grid=(S // SC7,),
        in_specs=[
            pl.BlockSpec((HB, SC7, 2 * D), lambda j: (0, j, 0)),
            pl.BlockSpec((SC7, 2 * D), lambda j: (j, 0)),
            pl.BlockSpec((1, 2 * D), lambda j: (0, 0)),
            pl.BlockSpec((1, 2 * D), lambda j: (0, 0)),
        ],
        out_specs=pl.BlockSpec((SC7, D, B), lambda j: (j, 0, 0)),
        out_shape=jax.ShapeDtypeStruct((S, D, B), jnp.float32),
    )(g, pe2, gamma2, beta2)


def kernel(input_ids, W, gamma, beta):
    w2 = _tc_repack(W)
    w_lin = w2.reshape(-1).reshape(VOCAB, D)
    g = _sc_gather(input_ids.astype(jnp.int32), w_lin)
    g = g.reshape(-1).reshape(HB, S, 2 * D)
    pe2 = jnp.concatenate([_PE8, _PE8], axis=1)
    gam2 = jnp.tile(gamma.reshape(1, D), (1, 2))
    bet2 = jnp.tile(beta.reshape(1, D), (1, 2))
    y = _tc_layernorm(g, jnp.asarray(pe2), gam2, bet2)
    return jnp.transpose(y, (2, 0, 1))


# speedup vs baseline: 1.7196x; 1.1547x over previous
import functools
import math

import jax
import jax.numpy as jnp
import numpy as np
from jax import lax
from jax.experimental import pallas as pl
from jax.experimental.pallas import tpu as pltpu
from jax.experimental.pallas import tpu_sc as plsc

VOCAB = 1000000
D = 64
MAXLEN = 2048
B = 4096
S = 200
EPS = 1e-12

NC = 2
NS = 16
NW = NC * NS
N = B * S
HB = B // 2
PER_W = N // NW
CHUNK = 128
N_CHUNKS = PER_W // CHUNK


def _pe_over_8():
    position = np.arange(0, S, dtype=np.float32)[:, None]
    div_term = np.exp(
        np.arange(0, D, 2, dtype=np.float32) * (-math.log(10000.0) / D))
    pe = np.zeros((S, D), dtype=np.float32)
    pe[:, 0::2] = np.sin(position * div_term)
    pe[:, 1::2] = np.cos(position * div_term)
    return pe / 8.0


_PE8 = _pe_over_8()

VB = 8192


def _conv_body(wt_ref, o_ref):
    x = wt_ref[...]
    xt = x.T.reshape(VB // 2, 2, D)
    o_ref[:, :D] = xt[:, 0, :]
    o_ref[:, D:] = xt[:, 1, :]


def _tc_repack(W):
    wt = W.T
    return pl.pallas_call(
        _conv_body,
        grid=(pl.cdiv(VOCAB, VB),),
        in_specs=[pl.BlockSpec((D, VB), lambda i: (0, i))],
        out_specs=pl.BlockSpec((VB // 2, 2 * D), lambda i: (i, 0)),
        out_shape=jax.ShapeDtypeStruct((VOCAB // 2, 2 * D), jnp.float32),
    )(wt)


IB = HB // NW


def _sc_gather(input_ids, w_lin):
    mesh = plsc.VectorSubcoreMesh(core_axis_name="c", subcore_axis_name="s")

    @functools.partial(
        pl.kernel,
        mesh=mesh,
        compiler_params=pltpu.CompilerParams(
            use_tc_tiling_on_sc=False, needs_layout_passes=False),
        out_type=jax.ShapeDtypeStruct((N, D), jnp.float32),
        scratch_types=[
            pltpu.VMEM((IB * S,), jnp.int32),
            pltpu.VMEM((IB * S,), jnp.int32),
            pltpu.VMEM((N_CHUNKS, CHUNK), jnp.int32),
            pltpu.VMEM((CHUNK, D), jnp.float32),
            pltpu.VMEM((CHUNK, D), jnp.float32),
            pltpu.SemaphoreType.DMA,
            pltpu.SemaphoreType.DMA,
        ],
    )
    def k(ids_hbm, w_hbm, out_hbm, ids_a, ids_b, idx_v, rows0, rows1,
          sem0, sem1):
        wid = lax.axis_index("s") * NC + lax.axis_index("c")
        base = wid * PER_W

        @pl.loop(0, IB)
        def _(j):
            pltpu.sync_copy(ids_hbm.at[wid * IB + j],
                            ids_a.at[pl.ds(j * S, S)])
            pltpu.sync_copy(ids_hbm.at[HB + wid * IB + j],
                            ids_b.at[pl.ds(j * S, S)])

        iota2 = lax.iota(jnp.int32, 16) * 2

        @pl.loop(0, IB * S // 16)
        def _(v):
            q = v * 16
            pos = iota2 + (q * 2)
            row = lax.shift_right_logical(pos, 7)
            col = lax.bitwise_and(pos, 127)
            xa = ids_a[pl.ds(q, 16)]
            plsc.store_scatter(idx_v, [row, col], xa)
            pos1 = pos + 1
            row1 = lax.shift_right_logical(pos1, 7)
            col1 = lax.bitwise_and(pos1, 127)
            xb = ids_b[pl.ds(q, 16)]
            plsc.store_scatter(idx_v, [row1, col1], xb)

        @pl.loop(0, N_CHUNKS, step=2)
        def _(c):
            cpa = pltpu.async_copy(w_hbm.at[idx_v.at[c]], rows0, sem0)
            cpb = pltpu.async_copy(w_hbm.at[idx_v.at[c + 1]], rows1, sem1)
            cpa.wait()
            pltpu.sync_copy(rows0, out_hbm.at[pl.ds(base + c * CHUNK, CHUNK)])
            cpb.wait()
            pltpu.sync_copy(
                rows1, out_hbm.at[pl.ds(base + (c + 1) * CHUNK, CHUNK)])

    return k(input_ids, w_lin)


SC7 = 8


def _ln_body(g_ref, pe2_ref, gam2_ref, bet2_ref, o_ref):
    x = g_ref[...] + pe2_ref[...][None, :, :]
    lane = lax.broadcasted_iota(jnp.int32, (1, 1, 2 * D), 2)
    m_a = (lane < D).astype(jnp.float32)
    s_all = jnp.sum(x, axis=-1, keepdims=True)
    s_a = jnp.sum(x * m_a, axis=-1, keepdims=True)
    mu = jnp.where(lane < D, s_a / D, (s_all - s_a) / D)
    xc = x - mu
    q = xc * xc
    q_all = jnp.sum(q, axis=-1, keepdims=True)
    q_a = jnp.sum(q * m_a, axis=-1, keepdims=True)
    var = jnp.where(lane < D, q_a / D, (q_all - q_a) / D)
    y = xc * lax.rsqrt(var + EPS / 64.0)
    y = y * gam2_ref[...][None, :, :] + bet2_ref[...][None, :, :]
    for st in range(SC7):
        t = y[:, st, :].T
        o_ref[st, :, :HB] = t[:D]
        o_ref[st, :, HB:] = t[D:]


def _tc_layernorm(g, pe2, gamma2, beta2):
    return pl.pallas_call(
        _ln_body,
        grid=(S // SC7,),
        in_specs=[
            pl.BlockSpec((HB, SC7, 2 * D), lambda j: (0, j, 0)),
            pl.BlockSpec((SC7, 2 * D), lambda j: (j, 0)),
            pl.BlockSpec((1, 2 * D), lambda j: (0, 0)),
            pl.BlockSpec((1, 2 * D), lambda j: (0, 0)),
        ],
        out_specs=pl.BlockSpec((SC7, D, B), lambda j: (j, 0, 0)),
        out_shape=jax.ShapeDtypeStruct((S, D, B), jnp.float32),
    )(g, pe2, gamma2, beta2)


def kernel(input_ids, W, gamma, beta):
    w2 = _tc_repack(W)
    w_lin = w2.reshape(-1).reshape(VOCAB, D)
    g = _sc_gather(input_ids.astype(jnp.int32), w_lin)
    g = g.reshape(-1).reshape(HB, S, 2 * D)
    pe2 = jnp.concatenate([_PE8, _PE8], axis=1)
    gam2 = jnp.tile(gamma.reshape(1, D), (1, 2))
    bet2 = jnp.tile(beta.reshape(1, D), (1, 2))
    y = _tc_layernorm(g, jnp.asarray(pe2), gam2, bet2)
    return jnp.transpose(y, (2, 0, 1))


# speedup vs baseline: 1.8260x; 1.0619x over previous
import functools
import math

import jax
import jax.numpy as jnp
import numpy as np
from jax import lax
from jax.experimental import pallas as pl
from jax.experimental.pallas import tpu as pltpu
from jax.experimental.pallas import tpu_sc as plsc

VOCAB = 1000000
D = 64
MAXLEN = 2048
B = 4096
S = 200
EPS = 1e-12

NC = 2
NS = 16
NW = NC * NS
N = B * S
HB = B // 2
PER_W = N // NW
CHUNK = 128
N_CHUNKS = PER_W // CHUNK


def _pe_over_8():
    position = np.arange(0, S, dtype=np.float32)[:, None]
    div_term = np.exp(
        np.arange(0, D, 2, dtype=np.float32) * (-math.log(10000.0) / D))
    pe = np.zeros((S, D), dtype=np.float32)
    pe[:, 0::2] = np.sin(position * div_term)
    pe[:, 1::2] = np.cos(position * div_term)
    return pe / 8.0


_PE8 = _pe_over_8()

VB = 8192


def _conv_body(wt_ref, o_ref):
    x = wt_ref[...]
    xt = x.T.reshape(VB // 2, 2, D)
    o_ref[:, :D] = xt[:, 0, :]
    o_ref[:, D:] = xt[:, 1, :]


def _tc_repack(W):
    wt = W.T
    return pl.pallas_call(
        _conv_body,
        grid=(pl.cdiv(VOCAB, VB),),
        in_specs=[pl.BlockSpec((D, VB), lambda i: (0, i))],
        out_specs=pl.BlockSpec((VB // 2, 2 * D), lambda i: (i, 0)),
        out_shape=jax.ShapeDtypeStruct((VOCAB // 2, 2 * D), jnp.float32),
    )(wt)


IB = HB // NW


def _sc_gather(input_ids, w_lin):
    mesh = plsc.VectorSubcoreMesh(core_axis_name="c", subcore_axis_name="s")

    @functools.partial(
        pl.kernel,
        mesh=mesh,
        compiler_params=pltpu.CompilerParams(
            use_tc_tiling_on_sc=False, needs_layout_passes=False),
        out_type=jax.ShapeDtypeStruct((N, D), jnp.float32),
        scratch_types=[
            pltpu.VMEM((IB, S), jnp.int32),
            pltpu.VMEM((IB, S), jnp.int32),
            pltpu.VMEM((N_CHUNKS, CHUNK), jnp.int32),
            pltpu.VMEM((CHUNK, D), jnp.float32),
            pltpu.VMEM((CHUNK, D), jnp.float32),
            pltpu.SemaphoreType.DMA,
            pltpu.SemaphoreType.DMA,
        ],
    )
    def k(ids_hbm, w_hbm, out_hbm, ids_a, ids_b, idx_v, rows0, rows1,
          sem0, sem1):
        wid = lax.axis_index("s") * NC + lax.axis_index("c")
        base = wid * PER_W

        cpi = pltpu.async_copy(ids_hbm.at[pl.ds(wid * IB, IB)], ids_a, sem0)
        pltpu.sync_copy(ids_hbm.at[pl.ds(HB + wid * IB, IB)], ids_b)
        cpi.wait()

        iota2 = lax.iota(jnp.int32, 16) * 2
        offsets = list(range(0, S - 15, 16)) + [S - 16]

        @pl.loop(0, IB)
        def _(j):
            pbase = j * (2 * S)
            for q in offsets:
                pos = iota2 + (pbase + q * 2)
                row = lax.shift_right_logical(pos, 7)
                col = lax.bitwise_and(pos, 127)
                xa = ids_a[j, pl.ds(q, 16)]
                plsc.store_scatter(idx_v, [row, col], xa)
                pos1 = pos + 1
                row1 = lax.shift_right_logical(pos1, 7)
                col1 = lax.bitwise_and(pos1, 127)
                xb = ids_b[j, pl.ds(q, 16)]
                plsc.store_scatter(idx_v, [row1, col1], xb)

        @pl.loop(0, N_CHUNKS, step=2)
        def _(c):
            cpa = pltpu.async_copy(w_hbm.at[idx_v.at[c]], rows0, sem0)
            cpb = pltpu.async_copy(w_hbm.at[idx_v.at[c + 1]], rows1, sem1)
            cpa.wait()
            pltpu.sync_copy(rows0, out_hbm.at[pl.ds(base + c * CHUNK, CHUNK)])
            cpb.wait()
            pltpu.sync_copy(
                rows1, out_hbm.at[pl.ds(base + (c + 1) * CHUNK, CHUNK)])

    return k(input_ids, w_lin)


SC7 = 8


def _ln_body(g_ref, pe2_ref, gam2_ref, bet2_ref, o_ref):
    x = g_ref[...] + pe2_ref[...][None, :, :]
    lane = lax.broadcasted_iota(jnp.int32, (1, 1, 2 * D), 2)
    m_a = (lane < D).astype(jnp.float32)
    s_all = jnp.sum(x, axis=-1, keepdims=True)
    s_a = jnp.sum(x * m_a, axis=-1, keepdims=True)
    mu = jnp.where(lane < D, s_a / D, (s_all - s_a) / D)
    xc = x - mu
    q = xc * xc
    q_all = jnp.sum(q, axis=-1, keepdims=True)
    q_a = jnp.sum(q * m_a, axis=-1, keepdims=True)
    var = jnp.where(lane < D, q_a / D, (q_all - q_a) / D)
    y = xc * lax.rsqrt(var + EPS / 64.0)
    y = y * gam2_ref[...][None, :, :] + bet2_ref[...][None, :, :]
    for st in range(SC7):
        t = y[:, st, :].T
        o_ref[st, :, :HB] = t[:D]
        o_ref[st, :, HB:] = t[D:]


def _tc_layernorm(g, pe2, gamma2, beta2):
    return pl.pallas_call(
        _ln_body,
        grid=(S // SC7,),
        in_specs=[
            pl.BlockSpec((HB, SC7, 2 * D), lambda j: (0, j, 0)),
            pl.BlockSpec((SC7, 2 * D), lambda j: (j, 0)),
            pl.BlockSpec((1, 2 * D), lambda j: (0, 0)),
            pl.BlockSpec((1, 2 * D), lambda j: (0, 0)),
        ],
        out_specs=pl.BlockSpec((SC7, D, B), lambda j: (j, 0, 0)),
        out_shape=jax.ShapeDtypeStruct((S, D, B), jnp.float32),
    )(g, pe2, gamma2, beta2)


def kernel(input_ids, W, gamma, beta):
    w2 = _tc_repack(W)
    w_lin = w2.reshape(-1).reshape(VOCAB, D)
    g = _sc_gather(input_ids.astype(jnp.int32), w_lin)
    g = g.reshape(-1).reshape(HB, S, 2 * D)
    pe2 = jnp.concatenate([_PE8, _PE8], axis=1)
    gam2 = jnp.tile(gamma.reshape(1, D), (1, 2))
    bet2 = jnp.tile(beta.reshape(1, D), (1, 2))
    y = _tc_layernorm(g, jnp.asarray(pe2), gam2, bet2)
    return jnp.transpose(y, (2, 0, 1))
